# baseline (speedup 1.0000x reference)
import jax
import jax.numpy as jnp
from jax import lax
from jax.experimental import pallas as pl
from jax.experimental.pallas import tpu as pltpu
from jax.experimental.pallas import tpu_sc as plsc

VEC = 768
LAT = 16384
K = 32
B = 128
LT = 2048
NT = LAT // LT
LTD = 2048
NTD = LAT // LTD

NWORK = 32
RPW = B // NWORK
NLANE = 16
NCHUNK = LAT // NLANE
NEG = -3.4e38


def _enc_body(x_ref, we_ref, be_ref, bd_ref, a_ref, t0_ref, m1_ref):
    t = pl.program_id(0)
    xbar = x_ref[...] - bd_ref[...]
    a = jax.lax.dot_general(
        xbar, we_ref[...], (((1,), (1,)), ((), ())),
        preferred_element_type=jnp.float32,
        precision=jax.lax.Precision.DEFAULT,
    )
    a = a + be_ref[...]
    a_ref[...] = a

    @pl.when(t == 0)
    def _():
        m1_ref[...] = jnp.full_like(m1_ref, NEG)

    m1 = m1_ref[...]
    for u in range(LT // 128):
        m1 = jnp.maximum(m1, a[:, u * 128:(u + 1) * 128])
    m1_ref[...] = m1

    @pl.when(t == NT - 1)
    def _():
        t0_ref[...] = m1


def _encode(x, W_enc, b_enc, b_dec):
    return pl.pallas_call(
        _enc_body,
        grid=(NT,),
        in_specs=[
            pl.BlockSpec((B, VEC), lambda t: (0, 0)),
            pl.BlockSpec((LT, VEC), lambda t: (t, 0)),
            pl.BlockSpec((1, LT), lambda t: (0, t)),
            pl.BlockSpec((1, VEC), lambda t: (0, 0)),
        ],
        out_specs=[
            pl.BlockSpec((B, LT), lambda t: (0, t)),
            pl.BlockSpec((B, 128), lambda t: (0, 0)),
        ],
        out_shape=[
            jax.ShapeDtypeStruct((B, LAT), jnp.float32),
            jax.ShapeDtypeStruct((B, 128), jnp.float32),
        ],
        scratch_shapes=[
            pltpu.VMEM((B, 128), jnp.float32),
        ],
        compiler_params=pltpu.CompilerParams(
            dimension_semantics=("arbitrary",),
        ),
    )(x, W_enc, b_enc.reshape(1, LAT), b_dec.reshape(1, VEC))


def _dec_body(a_ref, th_ref, wd_ref, bd_ref, f_ref, xhat_ref, acc_ref):
    t = pl.program_id(0)

    @pl.when(t == 0)
    def _():
        acc_ref[...] = jnp.zeros_like(acc_ref)

    a = a_ref[...]
    th = th_ref[...][:, :1]
    f = jnp.where((a >= th) & (a > 0.0), a, 0.0)
    f_ref[...] = f
    acc_ref[...] += jax.lax.dot_general(
        f, wd_ref[...], (((1,), (1,)), ((), ())),
        preferred_element_type=jnp.float32,
        precision=jax.lax.Precision.DEFAULT,
    )

    @pl.when(t == NTD - 1)
    def _():
        xhat_ref[...] = acc_ref[...] + bd_ref[...]


def _decode(a, thresh, W_dec, b_dec):
    return pl.pallas_call(
        _dec_body,
        grid=(NTD,),
        in_specs=[
            pl.BlockSpec((B, LTD), lambda t: (0, t)),
            pl.BlockSpec((B, NLANE), lambda t: (0, 0)),
            pl.BlockSpec((VEC, LTD), lambda t: (0, t)),
            pl.BlockSpec((1, VEC), lambda t: (0, 0)),
        ],
        out_specs=[
            pl.BlockSpec((B, LTD), lambda t: (0, t)),
            pl.BlockSpec((B, VEC), lambda t: (0, 0)),
        ],
        out_shape=[
            jax.ShapeDtypeStruct((B, LAT), jnp.float32),
            jax.ShapeDtypeStruct((B, VEC), jnp.float32),
        ],
        scratch_shapes=[pltpu.VMEM((B, VEC), jnp.float32)],
        compiler_params=pltpu.CompilerParams(
            dimension_semantics=("arbitrary",),
        ),
    )(a, thresh, W_dec, b_dec.reshape(1, VEC))


def _merge_top32(A, Bv, v_unsorted):
    vs = jnp.sort(v_unsorted)
    B2 = jnp.sort(jnp.maximum(Bv, jnp.flip(vs, 0)))
    rB2 = jnp.flip(B2, 0)
    newA = jnp.sort(jnp.maximum(A, rB2))
    newB = jnp.sort(jnp.minimum(A, rB2))
    return newA, newB


DEPTH = 1024
UNROLL = 16


def _sc_thresh_body(a_hbm, t0_hbm, out_hbm, rbuf0, rbuf1, rbuf2, rbuf3, cbuf,
                    tbuf, obuf, sem0, sem1, sem2, sem3, semt):
    wid = lax.axis_index("s") * 2 + lax.axis_index("c")
    sems = (sem0, sem1, sem2, sem3)
    bufs = (rbuf0, rbuf1, rbuf2, rbuf3)
    lane = lax.iota(jnp.int32, NLANE)
    col0 = lane * DEPTH
    neg = jnp.full((NLANE,), NEG, jnp.float32)

    cp_t = pltpu.async_copy(t0_hbm.at[pl.ds(RPW * wid, RPW)], tbuf, semt)
    cps = [pltpu.async_copy(a_hbm.at[RPW * wid + r], bufs[r], sems[r])
           for r in range(RPW)]
    cp_t.wait()
    for rl in range(RPW):
        cps[rl].wait()
        rb = bufs[rl]

        Am, Bm = neg, neg
        for c in range(128 // NLANE):
            Am, Bm = _merge_top32(Am, Bm, tbuf[rl, pl.ds(c * NLANE, NLANE)])
        T = jnp.min(Bm)

        @plsc.parallel_loop(0, NCHUNK, step=1, unroll=UNROLL, carry=col0)
        def scan(i, pos):
            v = rb[pl.ds(i * NLANE, NLANE)]
            msk = v >= T
            plsc.store_scatter(cbuf, [pos], v, mask=msk)
            return pos + msk.astype(jnp.int32)

        pos = scan

        cnt = pos - col0
        max_cnt = jnp.max(cnt)

        def sel(j, AB):
            g = plsc.load_gather(cbuf, [col0 + j])
            g = jnp.where(j < cnt, g, NEG)
            return _merge_top32(AB[0], AB[1], g)

        A, Bv = lax.fori_loop(0, max_cnt, sel, (neg, neg))
        obuf[rl, :] = jnp.full((NLANE,), jnp.min(Bv), jnp.float32)

    pltpu.sync_copy(obuf, out_hbm.at[pl.ds(RPW * wid, RPW)])


def _sc_thresh(a, t0):
    mesh = plsc.VectorSubcoreMesh(core_axis_name="c", subcore_axis_name="s")
    fn = pl.kernel(
        _sc_thresh_body,
        out_type=jax.ShapeDtypeStruct((B, NLANE), jnp.float32),
        mesh=mesh,
        scratch_types=[
            pltpu.VMEM((LAT,), jnp.float32),
            pltpu.VMEM((LAT,), jnp.float32),
            pltpu.VMEM((LAT,), jnp.float32),
            pltpu.VMEM((LAT,), jnp.float32),
            pltpu.VMEM((NLANE * DEPTH,), jnp.float32),
            pltpu.VMEM((RPW, 128), jnp.float32),
            pltpu.VMEM((RPW, NLANE), jnp.float32),
            pltpu.SemaphoreType.DMA,
            pltpu.SemaphoreType.DMA,
            pltpu.SemaphoreType.DMA,
            pltpu.SemaphoreType.DMA,
            pltpu.SemaphoreType.DMA,
        ],
        compiler_params=pltpu.CompilerParams(needs_layout_passes=False),
    )
    return fn(a, t0)


def kernel(x, W_enc, b_enc, W_dec, b_dec):
    a, t0 = _encode(x, W_enc, b_enc, b_dec)
    thresh = _sc_thresh(a, t0)
    f, xhat = _decode(a, thresh, W_dec, b_dec)
    return (f, xhat)


# speedup vs baseline: 1.0105x; 1.0105x over previous
import jax
import jax.numpy as jnp
from jax import lax
from jax.experimental import pallas as pl
from jax.experimental.pallas import tpu as pltpu
from jax.experimental.pallas import tpu_sc as plsc

VEC = 768
LAT = 16384
K = 32
B = 128
LT = 2048
NT = LAT // LT
LTD = 2048
NTD = LAT // LTD

NWORK = 32
RPW = B // NWORK
NLANE = 16
NCHUNK = LAT // NLANE
NEG = -3.4e38


def _enc_body(x_ref, we_ref, be_ref, bd_ref, a_ref, t0_ref, m1_ref):
    t = pl.program_id(0)
    xbar = x_ref[...] - bd_ref[...]
    a = jax.lax.dot_general(
        xbar, we_ref[...], (((1,), (1,)), ((), ())),
        preferred_element_type=jnp.float32,
        precision=jax.lax.Precision.DEFAULT,
    )
    a = a + be_ref[...]
    a_ref[...] = a

    @pl.when(t == 0)
    def _():
        m1_ref[...] = jnp.full_like(m1_ref, NEG)

    m1 = m1_ref[...]
    for u in range(LT // 128):
        m1 = jnp.maximum(m1, a[:, u * 128:(u + 1) * 128])
    m1_ref[...] = m1

    @pl.when(t == NT - 1)
    def _():
        t0_ref[...] = m1


def _encode(x, W_enc, b_enc, b_dec):
    return pl.pallas_call(
        _enc_body,
        grid=(NT,),
        in_specs=[
            pl.BlockSpec((B, VEC), lambda t: (0, 0)),
            pl.BlockSpec((LT, VEC), lambda t: (t, 0)),
            pl.BlockSpec((1, LT), lambda t: (0, t)),
            pl.BlockSpec((1, VEC), lambda t: (0, 0)),
        ],
        out_specs=[
            pl.BlockSpec((B, LT), lambda t: (0, t)),
            pl.BlockSpec((B, 128), lambda t: (0, 0)),
        ],
        out_shape=[
            jax.ShapeDtypeStruct((B, LAT), jnp.float32),
            jax.ShapeDtypeStruct((B, 128), jnp.float32),
        ],
        scratch_shapes=[
            pltpu.VMEM((B, 128), jnp.float32),
        ],
        compiler_params=pltpu.CompilerParams(
            dimension_semantics=("arbitrary",),
        ),
    )(x, W_enc, b_enc.reshape(1, LAT), b_dec.reshape(1, VEC))


def _dec_body(a_ref, th_ref, wd_ref, bd_ref, f_ref, xhat_ref, acc_ref):
    t = pl.program_id(0)

    @pl.when(t == 0)
    def _():
        acc_ref[...] = jnp.zeros_like(acc_ref)

    a = a_ref[...]
    th = th_ref[...][:, :1]
    f = jnp.where((a >= th) & (a > 0.0), a, 0.0)
    f_ref[...] = f
    acc_ref[...] += jax.lax.dot_general(
        f, wd_ref[...], (((1,), (1,)), ((), ())),
        preferred_element_type=jnp.float32,
        precision=jax.lax.Precision.DEFAULT,
    )

    @pl.when(t == NTD - 1)
    def _():
        xhat_ref[...] = acc_ref[...] + bd_ref[...]


def _decode(a, thresh, W_dec, b_dec):
    return pl.pallas_call(
        _dec_body,
        grid=(NTD,),
        in_specs=[
            pl.BlockSpec((B, LTD), lambda t: (0, t)),
            pl.BlockSpec((B, NLANE), lambda t: (0, 0)),
            pl.BlockSpec((VEC, LTD), lambda t: (0, t)),
            pl.BlockSpec((1, VEC), lambda t: (0, 0)),
        ],
        out_specs=[
            pl.BlockSpec((B, LTD), lambda t: (0, t)),
            pl.BlockSpec((B, VEC), lambda t: (0, 0)),
        ],
        out_shape=[
            jax.ShapeDtypeStruct((B, LAT), jnp.float32),
            jax.ShapeDtypeStruct((B, VEC), jnp.float32),
        ],
        scratch_shapes=[pltpu.VMEM((B, VEC), jnp.float32)],
        compiler_params=pltpu.CompilerParams(
            dimension_semantics=("arbitrary",),
        ),
    )(a, thresh, W_dec, b_dec.reshape(1, VEC))


def _merge_top32(A, Bv, v_unsorted):
    vs = jnp.sort(v_unsorted)
    B2 = jnp.sort(jnp.maximum(Bv, jnp.flip(vs, 0)))
    rB2 = jnp.flip(B2, 0)
    newA = jnp.sort(jnp.maximum(A, rB2))
    newB = jnp.sort(jnp.minimum(A, rB2))
    return newA, newB


DEPTH = 1024
UNROLL = 16


def _sc_thresh_body(a_hbm, t0_hbm, out_hbm, rbuf0, rbuf1, rbuf2, rbuf3, cbuf,
                    tbuf, obuf, sem0, sem1, sem2, sem3, semt):
    wid = lax.axis_index("s") * 2 + lax.axis_index("c")
    sems = (sem0, sem1, sem2, sem3)
    bufs = (rbuf0, rbuf1, rbuf2, rbuf3)
    lane = lax.iota(jnp.int32, NLANE)
    col0 = lane * DEPTH
    neg = jnp.full((NLANE,), NEG, jnp.float32)

    cp_t = pltpu.async_copy(t0_hbm.at[pl.ds(RPW * wid, RPW)], tbuf, semt)
    cps = [pltpu.async_copy(a_hbm.at[RPW * wid + r], bufs[r], sems[r])
           for r in range(RPW)]
    cp_t.wait()

    Ts = []
    for rl in range(RPW):
        Am, Bm = neg, neg
        for c in range(128 // NLANE):
            Am, Bm = _merge_top32(Am, Bm, tbuf[rl, pl.ds(c * NLANE, NLANE)])
        Ts.append(jnp.min(Bm))

    for rl in range(RPW):
        cps[rl].wait()
        rb = bufs[rl]
        T = Ts[rl]

        @plsc.parallel_loop(0, NCHUNK, step=1, unroll=UNROLL, carry=col0)
        def scan(i, pos):
            v = rb[pl.ds(i * NLANE, NLANE)]
            msk = v >= T
            plsc.store_scatter(cbuf, [pos], v, mask=msk)
            return pos + msk.astype(jnp.int32)

        pos = scan

        cnt = pos - col0
        max_cnt = jnp.max(cnt)

        def sel(j, AB):
            g = plsc.load_gather(cbuf, [col0 + j])
            g = jnp.where(j < cnt, g, NEG)
            return _merge_top32(AB[0], AB[1], g)

        A, Bv = lax.fori_loop(0, max_cnt, sel, (neg, neg))
        obuf[rl, :] = jnp.full((NLANE,), jnp.min(Bv), jnp.float32)

    pltpu.sync_copy(obuf, out_hbm.at[pl.ds(RPW * wid, RPW)])


def _sc_thresh(a, t0):
    mesh = plsc.VectorSubcoreMesh(core_axis_name="c", subcore_axis_name="s")
    fn = pl.kernel(
        _sc_thresh_body,
        out_type=jax.ShapeDtypeStruct((B, NLANE), jnp.float32),
        mesh=mesh,
        scratch_types=[
            pltpu.VMEM((LAT,), jnp.float32),
            pltpu.VMEM((LAT,), jnp.float32),
            pltpu.VMEM((LAT,), jnp.float32),
            pltpu.VMEM((LAT,), jnp.float32),
            pltpu.VMEM((NLANE * DEPTH,), jnp.float32),
            pltpu.VMEM((RPW, 128), jnp.float32),
            pltpu.VMEM((RPW, NLANE), jnp.float32),
            pltpu.SemaphoreType.DMA,
            pltpu.SemaphoreType.DMA,
            pltpu.SemaphoreType.DMA,
            pltpu.SemaphoreType.DMA,
            pltpu.SemaphoreType.DMA,
        ],
        compiler_params=pltpu.CompilerParams(needs_layout_passes=False),
    )
    return fn(a, t0)


def kernel(x, W_enc, b_enc, W_dec, b_dec):
    a, t0 = _encode(x, W_enc, b_enc, b_dec)
    thresh = _sc_thresh(a, t0)
    f, xhat = _decode(a, thresh, W_dec, b_dec)
    return (f, xhat)
